# Initial kernel scaffold; baseline (speedup 1.0000x reference)
import jax
import jax.numpy as jnp
from jax.experimental import pallas as pl


def kernel(logits, targets):
    raise NotImplementedError("write your pallas kernel here")


# speedup vs baseline: 1.5328x; 1.5328x over previous
import functools

import jax
import jax.numpy as jnp
from jax.experimental import pallas as pl


def _ce_body(tgt_ref, x_ref, out_ref, *, num_tokens, nblocks, vocab):
    x = x_ref[...]
    m = jnp.max(x, axis=-1, keepdims=True)
    s = jnp.sum(jnp.exp(x - m), axis=-1, keepdims=True)
    lse = m + jnp.log(s)

    t = tgt_ref[...]
    lane = jax.lax.broadcasted_iota(jnp.int32, x.shape, 1)
    tgt_logit = jnp.sum(jnp.where(lane == t, x, 0.0), axis=-1, keepdims=True)

    part = jnp.sum(lse - tgt_logit, keepdims=True).reshape(1, 1)

    i = pl.program_id(0)

    @pl.when(i == 0)
    def _init():
        out_ref[...] = jnp.zeros((1, 1), jnp.float32)

    out_ref[...] += part

    @pl.when(i == nblocks - 1)
    def _fin():
        out_ref[...] = out_ref[...] * (1.0 / num_tokens)


@functools.partial(jax.jit, static_argnames=("block_tokens",))
def _ce_mean(logits_flat, targets_col, block_tokens):
    num_tokens, vocab = logits_flat.shape
    nblocks = num_tokens // block_tokens
    body = functools.partial(
        _ce_body, num_tokens=num_tokens, nblocks=nblocks, vocab=vocab
    )
    out = pl.pallas_call(
        body,
        grid=(nblocks,),
        in_specs=[
            pl.BlockSpec((block_tokens, 1), lambda i: (i, 0)),
            pl.BlockSpec((block_tokens, vocab), lambda i: (i, 0)),
        ],
        out_specs=pl.BlockSpec((1, 1), lambda i: (0, 0)),
        out_shape=jax.ShapeDtypeStruct((1, 1), jnp.float32),
    )(targets_col, logits_flat)
    return out[0, 0]


def kernel(logits, targets):
    vocab = logits.shape[-1]
    logits_flat = logits.reshape(-1, vocab)
    targets_col = targets.reshape(-1, 1).astype(jnp.int32)
    return _ce_mean(logits_flat, targets_col, 16)


# speedup vs baseline: 1.9935x; 1.3006x over previous
import functools

import jax
import jax.numpy as jnp
from jax.experimental import pallas as pl


def _ce_body(tgt_ref, x_ref, out_ref, *, num_tokens, nblocks, vocab):
    x = x_ref[...]
    m = jnp.max(x, axis=-1, keepdims=True)
    s = jnp.sum(jnp.exp(x - m), axis=-1, keepdims=True)
    lse = m + jnp.log(s)

    t = tgt_ref[...]
    lane = jax.lax.broadcasted_iota(jnp.int32, x.shape, 1)
    tgt_logit = jnp.sum(jnp.where(lane == t, x, 0.0), axis=-1, keepdims=True)

    part = jnp.sum(lse - tgt_logit, keepdims=True).reshape(1, 1)

    i = pl.program_id(0)

    @pl.when(i == 0)
    def _init():
        out_ref[...] = jnp.zeros((1, 1), jnp.float32)

    out_ref[...] += part

    @pl.when(i == nblocks - 1)
    def _fin():
        out_ref[...] = out_ref[...] * (1.0 / num_tokens)


@functools.partial(jax.jit, static_argnames=("block_tokens",))
def _ce_mean(logits_flat, targets_col, block_tokens):
    num_tokens, vocab = logits_flat.shape
    nblocks = num_tokens // block_tokens
    body = functools.partial(
        _ce_body, num_tokens=num_tokens, nblocks=nblocks, vocab=vocab
    )
    out = pl.pallas_call(
        body,
        grid=(nblocks,),
        in_specs=[
            pl.BlockSpec((block_tokens, 1), lambda i: (i, 0)),
            pl.BlockSpec((block_tokens, vocab), lambda i: (i, 0)),
        ],
        out_specs=pl.BlockSpec((1, 1), lambda i: (0, 0)),
        out_shape=jax.ShapeDtypeStruct((1, 1), jnp.float32),
    )(targets_col, logits_flat)
    return out[0, 0]


def kernel(logits, targets):
    vocab = logits.shape[-1]
    logits_flat = logits.reshape(-1, vocab)
    targets_col = targets.reshape(-1, 1).astype(jnp.int32)
    return _ce_mean(logits_flat, targets_col, 32)


# speedup vs baseline: 2.5639x; 1.2862x over previous
import functools

import jax
import jax.numpy as jnp
from jax.experimental import pallas as pl


def _ce_body(tgt_ref, x_ref, out_ref, *, num_tokens, nblocks, vocab):
    x = x_ref[...]
    m = jnp.max(x, axis=-1, keepdims=True)
    s = jnp.sum(jnp.exp(x - m), axis=-1, keepdims=True)
    lse = m + jnp.log(s)

    t = tgt_ref[...]
    lane = jax.lax.broadcasted_iota(jnp.int32, x.shape, 1)
    tgt_logit = jnp.sum(jnp.where(lane == t, x, 0.0), axis=-1, keepdims=True)

    part = jnp.sum(lse - tgt_logit, keepdims=True).reshape(1, 1)

    i = pl.program_id(0)

    @pl.when(i == 0)
    def _init():
        out_ref[...] = jnp.zeros((1, 1), jnp.float32)

    out_ref[...] += part

    @pl.when(i == nblocks - 1)
    def _fin():
        out_ref[...] = out_ref[...] * (1.0 / num_tokens)


@functools.partial(jax.jit, static_argnames=("block_tokens",))
def _ce_mean(logits_flat, targets_col, block_tokens):
    num_tokens, vocab = logits_flat.shape
    nblocks = num_tokens // block_tokens
    body = functools.partial(
        _ce_body, num_tokens=num_tokens, nblocks=nblocks, vocab=vocab
    )
    out = pl.pallas_call(
        body,
        grid=(nblocks,),
        in_specs=[
            pl.BlockSpec((block_tokens, 1), lambda i: (i, 0)),
            pl.BlockSpec((block_tokens, vocab), lambda i: (i, 0)),
        ],
        out_specs=pl.BlockSpec((1, 1), lambda i: (0, 0)),
        out_shape=jax.ShapeDtypeStruct((1, 1), jnp.float32),
    )(targets_col, logits_flat)
    return out[0, 0]


def kernel(logits, targets):
    vocab = logits.shape[-1]
    logits_flat = logits.reshape(-1, vocab)
    targets_col = targets.reshape(-1, 1).astype(jnp.int32)
    return _ce_mean(logits_flat, targets_col, 64)


# speedup vs baseline: 2.7990x; 1.0917x over previous
import functools

import jax
import jax.numpy as jnp
from jax.experimental import pallas as pl


def _ce_body(tgt_ref, x_ref, out_ref, *, num_tokens, nblocks, vocab):
    x = x_ref[...]
    m = jnp.max(x, axis=-1, keepdims=True)
    s = jnp.sum(jnp.exp(x - m), axis=-1, keepdims=True)
    lse = m + jnp.log(s)

    t = tgt_ref[...]
    lane = jax.lax.broadcasted_iota(jnp.int32, x.shape, 1)
    tgt_logit = jnp.sum(jnp.where(lane == t, x, 0.0), axis=-1, keepdims=True)

    part = jnp.sum(lse - tgt_logit, keepdims=True).reshape(1, 1)

    i = pl.program_id(0)

    @pl.when(i == 0)
    def _init():
        out_ref[...] = jnp.zeros((1, 1), jnp.float32)

    out_ref[...] += part

    @pl.when(i == nblocks - 1)
    def _fin():
        out_ref[...] = out_ref[...] * (1.0 / num_tokens)


@functools.partial(jax.jit, static_argnames=("block_tokens",))
def _ce_mean(logits_flat, targets_col, block_tokens):
    num_tokens, vocab = logits_flat.shape
    nblocks = num_tokens // block_tokens
    body = functools.partial(
        _ce_body, num_tokens=num_tokens, nblocks=nblocks, vocab=vocab
    )
    out = pl.pallas_call(
        body,
        grid=(nblocks,),
        in_specs=[
            pl.BlockSpec((block_tokens, 1), lambda i: (i, 0)),
            pl.BlockSpec((block_tokens, vocab), lambda i: (i, 0)),
        ],
        out_specs=pl.BlockSpec((1, 1), lambda i: (0, 0)),
        out_shape=jax.ShapeDtypeStruct((1, 1), jnp.float32),
    )(targets_col, logits_flat)
    return out[0, 0]


def kernel(logits, targets):
    vocab = logits.shape[-1]
    logits_flat = logits.reshape(-1, vocab)
    targets_col = targets.reshape(-1, 1).astype(jnp.int32)
    return _ce_mean(logits_flat, targets_col, 128)
